# Initial kernel scaffold; baseline (speedup 1.0000x reference)
import jax
import jax.numpy as jnp
from jax.experimental import pallas as pl


def kernel(node_features, edge_index, edge_features, Wn, bn, Wq, bq, Wk, bk, Wv, bv, We, be, A1, b1, A2, b2, gamma, beta):
    raise NotImplementedError("write your pallas kernel here")


# speedup vs baseline: 2.5290x; 2.5290x over previous
import functools

import jax
import jax.numpy as jnp
from jax import lax
from jax.experimental import pallas as pl
from jax.experimental.pallas import tpu as pltpu
from jax.experimental.pallas import tpu_sc as plsc

NC = 2
NS = 16
NW = NC * NS
F32 = jnp.float32


def _dg(a, b, dims):
    return lax.dot_general(a, b, (dims, ((), ())), preferred_element_type=F32)


def _lrelu(x):
    return jnp.maximum(x, 0.2 * x)


def _project(x, Wn, bn, Wq, bq, Wk, bk, Wv, bv, A1):
    n, d = x.shape
    bn_ = n // 5

    def body(x_ref, wn_ref, bnr, wq_ref, bqr, wk_ref, bkr, wv_ref, bvr,
             a1_ref, h_ref, v_ref, qa_ref, ka_ref):
        xb = x_ref[...]
        a1 = a1_ref[...]
        h_ref[...] = _dg(xb, wn_ref[...], ((1,), (1,))) + bnr[...]
        v_ref[...] = _dg(xb, wv_ref[...], ((1,), (1,))) + bvr[...]
        tq = _dg(xb, wq_ref[...], ((1,), (1,))) + bqr[...]
        qa_ref[...] = _dg(tq, a1[:, :d], ((1,), (1,)))
        tk = _dg(xb, wk_ref[...], ((1,), (1,))) + bkr[...]
        ka_ref[...] = _dg(tk, a1[:, d:2 * d], ((1,), (1,)))

    row_spec = pl.BlockSpec((bn_, d), lambda i: (i, 0))
    full = lambda s: pl.BlockSpec(s, lambda i: (0, 0))
    out_sh = jax.ShapeDtypeStruct((n, d), F32)
    return pl.pallas_call(
        body,
        grid=(n // bn_,),
        in_specs=[row_spec, full((d, d)), full((1, d)), full((d, d)),
                  full((1, d)), full((d, d)), full((1, d)), full((d, d)),
                  full((1, d)), full((d, 3 * d))],
        out_specs=[row_spec] * 4,
        out_shape=[out_sh] * 4,
    )(x, Wn, bn, Wq, bq, Wk, bk, Wv, bv, A1)


def _edge_gather_sum(qa, ka, src, tgt):
    n, d = qa.shape
    e = src.shape[0]
    epw = e // NW
    ch = 80
    nch = epw // ch
    mesh = plsc.VectorSubcoreMesh(core_axis_name="c", subcore_axis_name="s",
                                  num_cores=NC, num_subcores=NS)

    @functools.partial(
        pl.kernel,
        out_type=jax.ShapeDtypeStruct((e, d), F32),
        mesh=mesh,
        scratch_types=[
            pltpu.VMEM((ch,), jnp.int32),
            pltpu.VMEM((ch,), jnp.int32),
            pltpu.VMEM((ch, d), F32),
            pltpu.VMEM((ch, d), F32),
            pltpu.SemaphoreType.DMA,
            pltpu.SemaphoreType.DMA,
        ])
    def k(qa_hbm, ka_hbm, src_hbm, tgt_hbm, out_hbm,
          sidx, tidx, qbuf, kbuf, sem1, sem2):
        wid = lax.axis_index("s") * NC + lax.axis_index("c")
        base0 = wid * epw

        def chunk(j, carry):
            base = base0 + j * ch
            pltpu.sync_copy(src_hbm.at[pl.ds(base, ch)], sidx)
            pltpu.sync_copy(tgt_hbm.at[pl.ds(base, ch)], tidx)
            cq = pltpu.async_copy(qa_hbm.at[sidx], qbuf, sem1)
            ck = pltpu.async_copy(ka_hbm.at[tidx], kbuf, sem2)
            cq.wait()
            ck.wait()

            def row(r, c2):
                for p in range(d // 16):
                    sl = pl.ds(16 * p, 16)
                    qbuf[r, sl] = qbuf[r, sl] + kbuf[r, sl]
                return c2

            lax.fori_loop(0, ch, row, 0)
            pltpu.sync_copy(qbuf, out_hbm.at[pl.ds(base, ch)])
            return carry

        lax.fori_loop(0, nch, chunk, 0)

    return k(qa, ka, src, tgt)


def _edge_scores(gsum, ef, We, be, A1, b1, A2, b2):
    e, d = gsum.shape
    de = ef.shape[1]
    nh = A2.shape[0]
    be_blk = 4000
    nb = e // be_blk

    def body(gs_ref, ef_ref, we_ref, ber, a1_ref, b1r, a2_ref, b2r,
             s_ref, stats_ref, msc, lsc):
        i = pl.program_id(0)
        a1e = a1_ref[...][:, 2 * d:]
        we2 = _dg(a1e, we_ref[...], ((1,), (0,)))
        c = _dg(ber[...], a1e, ((1,), (1,)))
        z = gs_ref[...] + _dg(ef_ref[...], we2, ((1,), (1,))) + c + b1r[...]
        z = _lrelu(z)
        t = _dg(z, a2_ref[...], ((1,), (1,))) + b2r[...]
        t = _lrelu(t)
        s = jnp.mean(t, axis=1, keepdims=True)
        s_ref[...] = s
        bm = jnp.max(s, keepdims=True)

        @pl.when(i == 0)
        def _():
            msc[...] = bm
            lsc[...] = jnp.sum(jnp.exp(s - bm), keepdims=True)

        @pl.when(i > 0)
        def _():
            m_old = msc[...]
            m_new = jnp.maximum(m_old, bm)
            lsc[...] = (lsc[...] * jnp.exp(m_old - m_new)
                        + jnp.sum(jnp.exp(s - m_new), keepdims=True))
            msc[...] = m_new

        @pl.when(i == nb - 1)
        def _():
            stats_ref[...] = jnp.concatenate([msc[...], lsc[...]], axis=1)

    full = lambda s: pl.BlockSpec(s, lambda i: (0, 0))
    return pl.pallas_call(
        body,
        grid=(nb,),
        in_specs=[pl.BlockSpec((be_blk, d), lambda i: (i, 0)),
                  pl.BlockSpec((be_blk, de), lambda i: (i, 0)),
                  full((d, de)), full((1, d)), full((d, 3 * d)),
                  full((1, d)), full((nh, d)), full((1, nh))],
        out_specs=[pl.BlockSpec((be_blk, 1), lambda i: (i, 0)),
                   full((1, 2))],
        out_shape=[jax.ShapeDtypeStruct((e, 1), F32),
                   jax.ShapeDtypeStruct((1, 2), F32)],
        scratch_shapes=[pltpu.VMEM((1, 1), F32), pltpu.VMEM((1, 1), F32)],
    )(gsum, ef, We, be, A1, b1, A2, b2)


def _softmax_weights(s, stats):
    e = s.shape[0]
    be_blk = 4000
    nb = e // be_blk

    def body(s_ref, st_ref, w_ref):
        m = st_ref[0:1, 0:1]
        l = st_ref[0:1, 1:2]
        w = jnp.exp(s_ref[...] - m) / l
        w_ref[...] = jnp.broadcast_to(w, (be_blk, 16))

    return pl.pallas_call(
        body,
        grid=(nb,),
        in_specs=[pl.BlockSpec((be_blk, 1), lambda i: (i, 0)),
                  pl.BlockSpec((1, 2), lambda i: (0, 0))],
        out_specs=pl.BlockSpec((be_blk, 16), lambda i: (i, 0)),
        out_shape=jax.ShapeDtypeStruct((e, 16), F32),
    )(s, stats)


def _scatter_agg(v, w16, src, tgt):
    n, d = v.shape
    e = src.shape[0]
    epw = e // NW
    ch = 80
    nch = epw // ch
    rows_a = (n // NS) // 8 * 8
    rem = n - NS * rows_a
    mesh = plsc.VectorSubcoreMesh(core_axis_name="c", subcore_axis_name="s",
                                  num_cores=NC, num_subcores=NS)

    @functools.partial(
        pl.kernel,
        out_type=jax.ShapeDtypeStruct((NC * n, d), F32),
        mesh=mesh,
        scratch_types=[
            pltpu.VMEM((ch,), jnp.int32),
            pltpu.VMEM((ch,), jnp.int32),
            pltpu.VMEM((ch, 16), F32),
            pltpu.VMEM((ch, d), F32),
            pltpu.VMEM((16, d), F32),
            pltpu.VMEM_SHARED((n, d), F32),
            pltpu.SemaphoreType.DMA,
        ])
    def k(v_hbm, w_hbm, src_hbm, tgt_hbm, out_hbm,
          sidx, tidx, wbuf, vbuf, zbuf, accum, sem):
        cid = lax.axis_index("c")
        sid = lax.axis_index("s")
        wid = sid * NC + cid

        def zrow(r, c2):
            for p in range(d // 16):
                zbuf[r, pl.ds(16 * p, 16)] = jnp.zeros((16,), F32)
            return c2

        lax.fori_loop(0, 16, zrow, 0)
        row0 = sid * rows_a

        def zchunk(i, c2):
            pltpu.sync_copy(zbuf, accum.at[pl.ds(row0 + i * 16, 16)])
            return c2

        lax.fori_loop(0, rows_a // 16, zchunk, 0)

        @pl.when(sid == NS - 1)
        def _():
            pltpu.sync_copy(zbuf, accum.at[pl.ds(NS * rows_a, rem)])

        plsc.subcore_barrier()
        base0 = wid * epw

        def chunk(j, carry):
            base = base0 + j * ch
            pltpu.sync_copy(src_hbm.at[pl.ds(base, ch)], sidx)
            pltpu.sync_copy(tgt_hbm.at[pl.ds(base, ch)], tidx)
            pltpu.sync_copy(w_hbm.at[pl.ds(base, ch)], wbuf)
            pltpu.async_copy(v_hbm.at[tidx], vbuf, sem).wait()

            def row(r, c2):
                wv = wbuf[r, pl.ds(0, 16)]
                for p in range(d // 16):
                    sl = pl.ds(16 * p, 16)
                    vbuf[r, sl] = vbuf[r, sl] * wv
                return c2

            lax.fori_loop(0, ch, row, 0)
            pltpu.sync_copy(vbuf, accum.at[sidx], add=True)
            return carry

        lax.fori_loop(0, nch, chunk, 0)
        plsc.subcore_barrier()
        pltpu.sync_copy(accum.at[pl.ds(row0, rows_a)],
                        out_hbm.at[pl.ds(cid * n + row0, rows_a)])

        @pl.when(sid == NS - 1)
        def _():
            pltpu.sync_copy(accum.at[pl.ds(NS * rows_a, rem)],
                            out_hbm.at[pl.ds(cid * n + NS * rows_a, rem)])

    return k(v, w16, src, tgt)


def _finalize(h, a0, a1, gamma, beta):
    n, d = h.shape
    bn_ = n // 5

    def body(h_ref, a0_ref, a1_ref, g_ref, b_ref, o_ref):
        y = h_ref[...] + a0_ref[...] + a1_ref[...]
        mu = jnp.mean(y, axis=1, keepdims=True)
        yc = y - mu
        var = jnp.mean(yc * yc, axis=1, keepdims=True)
        o_ref[...] = yc * lax.rsqrt(var + 1e-5) * g_ref[...] + b_ref[...]

    row_spec = pl.BlockSpec((bn_, d), lambda i: (i, 0))
    full = lambda s: pl.BlockSpec(s, lambda i: (0, 0))
    return pl.pallas_call(
        body,
        grid=(n // bn_,),
        in_specs=[row_spec, row_spec, row_spec, full((1, d)), full((1, d))],
        out_specs=row_spec,
        out_shape=jax.ShapeDtypeStruct((n, d), F32),
    )(h, a0, a1, gamma, beta)


def kernel(node_features, edge_index, edge_features, Wn, bn, Wq, bq, Wk, bk,
           Wv, bv, We, be, A1, b1, A2, b2, gamma, beta):
    n, d = node_features.shape
    src = edge_index[0]
    tgt = edge_index[1]
    r = lambda x: x[None, :]
    h, v, qa, ka = _project(node_features, Wn, r(bn), Wq, r(bq), Wk, r(bk),
                            Wv, r(bv), A1)
    gsum = _edge_gather_sum(qa, ka, src, tgt)
    s, stats = _edge_scores(gsum, edge_features, We, r(be), A1, r(b1), A2,
                            r(b2))
    w16 = _softmax_weights(s, stats)
    aggp = _scatter_agg(v, w16, src, tgt)
    return _finalize(h, aggp[:n], aggp[n:], r(gamma), r(beta))


# speedup vs baseline: 3.0295x; 1.1979x over previous
import functools

import jax
import jax.numpy as jnp
from jax import lax
from jax.experimental import pallas as pl
from jax.experimental.pallas import tpu as pltpu
from jax.experimental.pallas import tpu_sc as plsc

NC = 2
NS = 16
CH = 80
F32 = jnp.float32


def _dg(a, b, dims):
    return lax.dot_general(a, b, (dims, ((), ())), preferred_element_type=F32)


def _lrelu(x):
    return jnp.maximum(x, 0.2 * x)


def _project(x, Wn, bn, Wq, bq, Wk, bk, Wv, bv, A1):
    n, d = x.shape
    dh = d // 2
    bn_ = n // 5

    def body(x_ref, wn_ref, bnr, wq_ref, bqr, wk_ref, bkr, wv_ref, bvr,
             a1_ref, h_ref, v_ref, qa0_ref, qa1_ref, ka_ref):
        xb = x_ref[...]
        a1 = a1_ref[...]
        h_ref[...] = _dg(xb, wn_ref[...], ((1,), (1,))) + bnr[...]
        v_ref[...] = _dg(xb, wv_ref[...], ((1,), (1,))) + bvr[...]
        tq = _dg(xb, wq_ref[...], ((1,), (1,))) + bqr[...]
        qa = _dg(tq, a1[:, :d], ((1,), (1,)))
        qa0_ref[...] = qa[:, :dh]
        qa1_ref[...] = qa[:, dh:]
        tk = _dg(xb, wk_ref[...], ((1,), (1,))) + bkr[...]
        ka_ref[...] = _dg(tk, a1[:, d:2 * d], ((1,), (1,)))

    row_spec = pl.BlockSpec((bn_, d), lambda i: (i, 0))
    half_spec = pl.BlockSpec((bn_, dh), lambda i: (i, 0))
    full = lambda s: pl.BlockSpec(s, lambda i: (0, 0))
    half_sh = jax.ShapeDtypeStruct((n, dh), F32)
    return pl.pallas_call(
        body,
        grid=(n // bn_,),
        in_specs=[row_spec, full((d, d)), full((1, d)), full((d, d)),
                  full((1, d)), full((d, d)), full((1, d)), full((d, d)),
                  full((1, d)), full((d, 3 * d))],
        out_specs=[row_spec, row_spec] + [half_spec] * 2 + [row_spec],
        out_shape=([jax.ShapeDtypeStruct((n, d), F32)] * 2 + [half_sh] * 2
                   + [jax.ShapeDtypeStruct((n, d), F32)]),
    )(x, Wn, bn, Wq, bq, Wk, bk, Wv, bv, A1)


def _edge_gather_sum(qaf, kaf, idx3):
    n, d = qaf.shape
    dh = d // 2
    nrows, _, ch = idx3.shape
    e = nrows * ch
    ept = e // NS
    nch = ept // ch
    mesh = plsc.VectorSubcoreMesh(core_axis_name="c", subcore_axis_name="s",
                                  num_cores=NC, num_subcores=NS)

    @functools.partial(
        pl.kernel,
        out_type=jax.ShapeDtypeStruct((NC * e, dh), F32),
        mesh=mesh,
        scratch_types=[
            pltpu.VMEM((2, CH), jnp.int32), pltpu.VMEM((2, CH), jnp.int32),
            pltpu.VMEM((CH, 128), F32), pltpu.VMEM((CH, 128), F32),
            pltpu.VMEM((CH, 128), F32), pltpu.VMEM((CH, 128), F32),
            pltpu.VMEM((CH, 64), F32), pltpu.VMEM((CH, 64), F32),
            pltpu.SemaphoreType.DMA, pltpu.SemaphoreType.DMA,
            pltpu.SemaphoreType.DMA, pltpu.SemaphoreType.DMA,
            pltpu.SemaphoreType.DMA, pltpu.SemaphoreType.DMA,
            pltpu.SemaphoreType.DMA, pltpu.SemaphoreType.DMA,
        ])
    def k(qaf_h, kaf_h, idx_h, out_h,
          is0, is1, qb0, qb1, kb0, kb1, ob0, ob1,
          si0, si1, sq0, sq1, sk0, sk1, so0, so1):
        cid = lax.axis_index("c")
        sid = lax.axis_index("s")

        irow0 = sid * nch
        base_out = cid * e + sid * ept
        coff = cid * dh
        bufs = [(is0, qb0, kb0, ob0, si0, sq0, sk0, so0),
                (is1, qb1, kb1, ob1, si1, sq1, sk1, so1)]

        def idx_load(j, b):
            isb = bufs[b][0]
            pltpu.async_copy(idx_h.at[irow0 + j], isb, bufs[b][5])

        def idx_wait(j, b):
            isb = bufs[b][0]
            pltpu.make_async_copy(idx_h.at[irow0 + j], isb, bufs[b][5]).wait()

        def start_gathers(b):
            isb, qb, kb, _, _, sq, sk, _ = bufs[b]
            pltpu.async_copy(qaf_h.at[isb.at[0]], qb, sq)
            pltpu.async_copy(kaf_h.at[isb.at[1]], kb, sk)

        for b in range(2):
            idx_load(b, b)
            idx_wait(b, b)
            start_gathers(b)

        def pair(j2, carry):
            for b in range(2):
                isb, qb, kb, ob, si, sq, sk, so = bufs[b]
                j = j2 * 2 + b
                pltpu.make_async_copy(qaf_h.at[isb.at[0]], qb, sq).wait()
                pltpu.make_async_copy(kaf_h.at[isb.at[1]], kb, sk).wait()

                @pl.when(j + 2 < nch)
                def _():
                    idx_load(j + 2, b)

                @pl.when(j >= 2)
                def _():
                    pltpu.make_async_copy(
                        ob, out_h.at[pl.ds(base_out + (j - 2) * ch, ch)],
                        so).wait()

                def row(r, c2):
                    for p in range(dh // 16):
                        sl = pl.ds(16 * p, 16)
                        ksl = pl.ds(coff + 16 * p, 16)
                        ob[r, sl] = qb[r, ksl] + kb[r, ksl]
                    return c2

                lax.fori_loop(0, ch, row, 0)
                pltpu.async_copy(
                    ob, out_h.at[pl.ds(base_out + j * ch, ch)], so)

                @pl.when(j + 2 < nch)
                def _():
                    idx_wait(j + 2, b)
                    start_gathers(b)
            return carry

        lax.fori_loop(0, nch // 2, pair, 0)
        for b in range(2):
            ob, so = bufs[b][3], bufs[b][7]
            j = nch - 2 + b
            pltpu.make_async_copy(
                ob, out_h.at[pl.ds(base_out + j * ch, ch)], so).wait()

    return k(qaf, kaf, idx3)


def _edge_scores(gs, ef, We, be, A1, b1, A2, b2):
    e2, dh = gs.shape
    e = e2 // 2
    d = 2 * dh
    de = ef.shape[1]
    nh = A2.shape[0]
    be_blk = 4000
    nb = e // be_blk

    def body(gs0_ref, gs1_ref, ef_ref, we_ref, ber, a1_ref, b1r, a2_ref,
             b2r, s_ref, stats_ref, msc, lsc):
        i = pl.program_id(0)
        a1e = a1_ref[...][:, 2 * d:]
        we2 = _dg(a1e, we_ref[...], ((1,), (0,)))
        c = _dg(ber[...], a1e, ((1,), (1,)))
        gsum = jnp.concatenate([gs0_ref[...], gs1_ref[...]], axis=1)
        z = gsum + _dg(ef_ref[...], we2, ((1,), (1,))) + c + b1r[...]
        z = _lrelu(z)
        t = _dg(z, a2_ref[...], ((1,), (1,))) + b2r[...]
        t = _lrelu(t)
        s = jnp.mean(t, axis=1, keepdims=True)
        s_ref[...] = s
        bm = jnp.max(s, keepdims=True)

        @pl.when(i == 0)
        def _():
            msc[...] = bm
            lsc[...] = jnp.sum(jnp.exp(s - bm), keepdims=True)

        @pl.when(i > 0)
        def _():
            m_old = msc[...]
            m_new = jnp.maximum(m_old, bm)
            lsc[...] = (lsc[...] * jnp.exp(m_old - m_new)
                        + jnp.sum(jnp.exp(s - m_new), keepdims=True))
            msc[...] = m_new

        @pl.when(i == nb - 1)
        def _():
            stats_ref[...] = jnp.concatenate([msc[...], lsc[...]], axis=1)

    full = lambda s: pl.BlockSpec(s, lambda i: (0, 0))
    return pl.pallas_call(
        body,
        grid=(nb,),
        in_specs=[pl.BlockSpec((be_blk, dh), lambda i: (i, 0)),
                  pl.BlockSpec((be_blk, dh), lambda i: (i + nb, 0)),
                  pl.BlockSpec((be_blk, de), lambda i: (i, 0)),
                  full((d, de)), full((1, d)), full((d, 3 * d)),
                  full((1, d)), full((nh, d)), full((1, nh))],
        out_specs=[pl.BlockSpec((be_blk, 1), lambda i: (i, 0)),
                   full((1, 2))],
        out_shape=[jax.ShapeDtypeStruct((e, 1), F32),
                   jax.ShapeDtypeStruct((1, 2), F32)],
        scratch_shapes=[pltpu.VMEM((1, 1), F32), pltpu.VMEM((1, 1), F32)],
    )(gs, gs, ef, We, be, A1, b1, A2, b2)


def _softmax_weights(s, stats):
    e = s.shape[0]
    be_blk = 4000
    nb = e // be_blk

    def body(s_ref, st_ref, w_ref):
        m = st_ref[0:1, 0:1]
        l = st_ref[0:1, 1:2]
        w = jnp.exp(s_ref[...] - m) / l
        w_ref[...] = jnp.broadcast_to(w, (be_blk, 16))

    return pl.pallas_call(
        body,
        grid=(nb,),
        in_specs=[pl.BlockSpec((be_blk, 1), lambda i: (i, 0)),
                  pl.BlockSpec((1, 2), lambda i: (0, 0))],
        out_specs=pl.BlockSpec((be_blk, 16), lambda i: (i, 0)),
        out_shape=jax.ShapeDtypeStruct((e, 16), F32),
    )(s, stats)


def _scatter_agg(vf, w16, src1d, tgt1d):
    n, d = vf.shape
    e = src1d.shape[0]
    eps = e // NC
    nch_a = (eps // NS) // CH // 2 * 2 + 2
    nch_l = eps // CH - (NS - 1) * nch_a
    rows_a = (n // NS) // 8 * 8
    rem = n - NS * rows_a
    mesh = plsc.VectorSubcoreMesh(core_axis_name="c", subcore_axis_name="s",
                                  num_cores=NC, num_subcores=NS)

    @functools.partial(
        pl.kernel,
        out_type=jax.ShapeDtypeStruct((NC * n, d), F32),
        mesh=mesh,
        scratch_types=[
            pltpu.VMEM((CH,), jnp.int32), pltpu.VMEM((CH,), jnp.int32),
            pltpu.VMEM((CH,), jnp.int32), pltpu.VMEM((CH,), jnp.int32),
            pltpu.VMEM((CH, 16), F32), pltpu.VMEM((CH, 16), F32),
            pltpu.VMEM((CH, 128), F32), pltpu.VMEM((CH, 128), F32),
            pltpu.VMEM((8, 128), F32),
            pltpu.VMEM_SHARED((n, d), F32),
            pltpu.SemaphoreType.DMA, pltpu.SemaphoreType.DMA,
            pltpu.SemaphoreType.DMA, pltpu.SemaphoreType.DMA,
            pltpu.SemaphoreType.DMA, pltpu.SemaphoreType.DMA,
            pltpu.SemaphoreType.DMA, pltpu.SemaphoreType.DMA,
        ])
    def k(vf_h, w_h, src_h, tgt_h, out_h,
          sx0, sx1, tx0, tx1, wb0, wb1, vb0, vb1, zbuf, accum,
          sa0, sa1, sb0, sb1, sv0, sv1, sw0, sw1):
        cid = lax.axis_index("c")
        sid = lax.axis_index("s")

        def zrow(r, c2):
            for p in range(d // 16):
                zbuf[r, pl.ds(16 * p, 16)] = jnp.zeros((16,), F32)
            return c2

        lax.fori_loop(0, 8, zrow, 0)
        row0 = sid * rows_a

        def zchunk(i, c2):
            pltpu.sync_copy(zbuf, accum.at[pl.ds(row0 + i * 8, 8)])
            return c2

        lax.fori_loop(0, rows_a // 8, zchunk, 0)

        @pl.when(sid == NS - 1)
        def _():
            pltpu.sync_copy(zbuf, accum.at[pl.ds(NS * rows_a, 8)])
            pltpu.sync_copy(zbuf, accum.at[pl.ds(NS * rows_a + 8, 8)])

        plsc.subcore_barrier()

        nch = jnp.where(sid < NS - 1, nch_a, nch_l)
        base_e = (cid * eps
                  + jnp.where(sid < NS - 1, sid * nch_a, (NS - 1) * nch_a)
                  * CH)
        bufs = [(sx0, tx0, wb0, vb0, sa0, sb0, sv0, sw0),
                (sx1, tx1, wb1, vb1, sa1, sb1, sv1, sw1)]

        def idx_load(j, b):
            sx, tx, _, _, sa, sb, _, _ = bufs[b]
            pltpu.async_copy(src_h.at[pl.ds(base_e + j * CH, CH)], sx, sa)
            pltpu.async_copy(tgt_h.at[pl.ds(base_e + j * CH, CH)], tx, sb)

        def idx_wait(j, b):
            sx, tx, _, _, sa, sb, _, _ = bufs[b]
            pltpu.make_async_copy(
                src_h.at[pl.ds(base_e + j * CH, CH)], sx, sa).wait()
            pltpu.make_async_copy(
                tgt_h.at[pl.ds(base_e + j * CH, CH)], tx, sb).wait()

        def start_loads(j, b):
            _, tx, wb, vb, _, _, sv, sw = bufs[b]
            pltpu.async_copy(vf_h.at[tx], vb, sv)
            pltpu.async_copy(w_h.at[pl.ds(base_e + j * CH, CH)], wb, sw)

        for b in range(2):
            idx_load(b, b)
            idx_wait(b, b)
            start_loads(b, b)

        def pair(j2, carry):
            for b in range(2):
                sx, tx, wb, vb, sa, sb, sv, sw = bufs[b]
                j = j2 * 2 + b
                pltpu.make_async_copy(vf_h.at[tx], vb, sv).wait()
                pltpu.make_async_copy(
                    w_h.at[pl.ds(base_e + j * CH, CH)], wb, sw).wait()

                def row(r, c2):
                    wv = wb[r, pl.ds(0, 16)]
                    for p in range(d // 16):
                        sl = pl.ds(16 * p, 16)
                        vb[r, sl] = vb[r, sl] * wv
                    return c2

                lax.fori_loop(0, CH, row, 0)
                pltpu.sync_copy(vb, accum.at[sx], add=True)

                @pl.when(j + 2 < nch)
                def _():
                    idx_load(j + 2, b)
                    idx_wait(j + 2, b)
                    start_loads(j + 2, b)
            return carry

        lax.fori_loop(0, nch // 2, pair, 0)
        plsc.subcore_barrier()
        pltpu.sync_copy(accum.at[pl.ds(row0, rows_a)],
                        out_h.at[pl.ds(cid * n + row0, rows_a)])

        @pl.when(sid == NS - 1)
        def _():
            pltpu.sync_copy(accum.at[pl.ds(NS * rows_a, rem)],
                            out_h.at[pl.ds(cid * n + NS * rows_a, rem)])

    return k(vf, w16, src1d, tgt1d)


def _finalize(h, agg, gamma, beta):
    n, d = h.shape
    bn_ = n // 5
    nbn = n // bn_

    def body(h_ref, a0_ref, a1_ref, g_ref, b_ref, o_ref):
        y = h_ref[...] + a0_ref[...] + a1_ref[...]
        mu = jnp.mean(y, axis=1, keepdims=True)
        yc = y - mu
        var = jnp.mean(yc * yc, axis=1, keepdims=True)
        o_ref[...] = yc * lax.rsqrt(var + 1e-5) * g_ref[...] + b_ref[...]

    row_spec = pl.BlockSpec((bn_, d), lambda i: (i, 0))
    full = lambda s: pl.BlockSpec(s, lambda i: (0, 0))
    return pl.pallas_call(
        body,
        grid=(nbn,),
        in_specs=[row_spec,
                  pl.BlockSpec((bn_, d), lambda i: (i, 0)),
                  pl.BlockSpec((bn_, d), lambda i: (i + nbn, 0)),
                  full((1, d)), full((1, d))],
        out_specs=row_spec,
        out_shape=jax.ShapeDtypeStruct((n, d), F32),
    )(h, agg, agg, gamma, beta)


def kernel(node_features, edge_index, edge_features, Wn, bn, Wq, bq, Wk, bk,
           Wv, bv, We, be, A1, b1, A2, b2, gamma, beta):
    idx3 = jnp.stack([edge_index[0].reshape(-1, CH),
                      edge_index[1].reshape(-1, CH)], axis=1)
    r = lambda x: x[None, :]
    h, vf, qa0, qa1, kaf = _project(
        node_features, Wn, r(bn), Wq, r(bq), Wk, r(bk), Wv, r(bv), A1)
    qaf = jnp.concatenate([qa0, qa1], axis=1)
    gs = _edge_gather_sum(qaf, kaf, idx3)
    s, stats = _edge_scores(gs, edge_features, We, r(be), A1, r(b1), A2,
                            r(b2))
    w16 = _softmax_weights(s, stats)
    agg = _scatter_agg(vf, w16, edge_index[0], edge_index[1])
    return _finalize(h, agg, r(gamma), r(beta))
